# Initial kernel scaffold; baseline (speedup 1.0000x reference)
import jax
import jax.numpy as jnp
from jax.experimental import pallas as pl


def kernel(x, codebooks):
    raise NotImplementedError("write your pallas kernel here")


# speedup vs baseline: 1.1144x; 1.1144x over previous
import jax
import jax.numpy as jnp
from jax.experimental import pallas as pl
from jax.experimental.pallas import tpu as pltpu

N_Q = 8
BINS = 1024
DIM = 256
B = 8
T = 2048
NTOK = B * T
TOK = 2048
NB = NTOK // TOK


def _stage_kernel(r_ref, rn_ref, cb_ref, cbn_ref, quant_ref,
                  r_out_ref, quant_out_ref, codes_ref, loss_ref):
    j = pl.program_id(0)

    @pl.when(j == 0)
    def _init():
        loss_ref[...] = jnp.zeros_like(loss_ref)

    r = r_ref[...]
    cb = cb_ref[...]
    rn = rn_ref[0]
    s = jax.lax.dot_general(
        r, cb, (((1,), (1,)), ((), ())),
        preferred_element_type=jnp.float32,
    )
    d = rn.reshape(TOK, 1) - 2.0 * s + cbn_ref[...]
    idx = jnp.argmin(d, axis=1)
    onehot = (
        jax.lax.broadcasted_iota(jnp.int32, (TOK, BINS), 1) == idx[:, None]
    ).astype(jnp.float32)
    q = jax.lax.dot_general(
        onehot, cb, (((1,), (0,)), ((), ())),
        preferred_element_type=jnp.float32,
        precision=jax.lax.Precision.HIGHEST,
    )
    diff = q - r
    loss_ref[0:1, :] += jnp.sum(diff * diff)
    q_st = r + diff
    quant_out_ref[...] = quant_ref[...] + q_st
    r_out_ref[...] = r - q_st
    codes_ref[0, 0, :] = idx


def _stage(residual, rn, cb, cbn, quant):
    return pl.pallas_call(
        _stage_kernel,
        grid=(NB,),
        in_specs=[
            pl.BlockSpec((TOK, DIM), lambda j: (j, 0)),
            pl.BlockSpec((1, 1, TOK), lambda j: (j, 0, 0)),
            pl.BlockSpec((BINS, DIM), lambda j: (0, 0)),
            pl.BlockSpec((1, BINS), lambda j: (0, 0)),
            pl.BlockSpec((TOK, DIM), lambda j: (j, 0)),
        ],
        out_specs=[
            pl.BlockSpec((TOK, DIM), lambda j: (j, 0)),
            pl.BlockSpec((TOK, DIM), lambda j: (j, 0)),
            pl.BlockSpec((1, 1, TOK), lambda j: (j, 0, 0)),
            pl.BlockSpec((8, 128), lambda j: (0, 0)),
        ],
        out_shape=[
            jax.ShapeDtypeStruct((NTOK, DIM), jnp.float32),
            jax.ShapeDtypeStruct((NTOK, DIM), jnp.float32),
            jax.ShapeDtypeStruct((NB, 1, TOK), jnp.int32),
            jax.ShapeDtypeStruct((8, 128), jnp.float32),
        ],
        input_output_aliases={0: 0, 4: 1},
    )(residual, rn, cb, cbn, quant)


def kernel(x, codebooks):
    xt = jnp.transpose(x, (0, 2, 1))
    cbn = jnp.sum(codebooks * codebooks, axis=-1)

    residual = xt.reshape(NTOK, DIM)
    quant = jnp.zeros((NTOK, DIM), jnp.float32)
    codes_list = []
    loss_list = []
    for i in range(N_Q):
        rn = jnp.sum(
            residual.reshape(B, T, DIM) * residual.reshape(B, T, DIM),
            axis=-1, keepdims=True,
        ).reshape(NB, 1, TOK)
        residual, quant, codes_i, loss_i = _stage(
            residual, rn, codebooks[i], cbn[i].reshape(1, BINS), quant
        )
        codes_list.append(codes_i.reshape(NTOK))
        loss_list.append(loss_i[0, 0])

    quantized = jnp.transpose(quant.reshape(B, T, DIM), (0, 2, 1))
    codes = jnp.stack(codes_list, axis=0).reshape(N_Q, B, T).astype(jnp.int64)
    commit_loss = jnp.mean(
        jnp.stack(loss_list) / jnp.float32(NTOK * DIM)
    )
    return quantized, codes, commit_loss


# speedup vs baseline: 1.3453x; 1.2072x over previous
import jax
import jax.numpy as jnp
from jax.experimental import pallas as pl
from jax.experimental.pallas import tpu as pltpu

N_Q = 8
BINS = 1024
DIM = 256
B = 8
T = 2048
NTOK = B * T
TOK = 2048
NB = NTOK // TOK


def _stage_kernel(r_ref, rn_ref, cb_ref, c1_ref, c2_ref, c3_ref, cbn_ref,
                  quant_ref, r_out_ref, quant_out_ref, codes_ref, loss_ref):
    j = pl.program_id(0)

    @pl.when(j == 0)
    def _init():
        loss_ref[...] = jnp.zeros_like(loss_ref)

    r = r_ref[...]
    cb = cb_ref[...]
    rn = rn_ref[0]
    s = jax.lax.dot_general(
        r, cb, (((1,), (1,)), ((), ())),
        preferred_element_type=jnp.float32,
    )
    d = rn.reshape(TOK, 1) - 2.0 * s + cbn_ref[...]
    idx = jnp.argmin(d, axis=1)
    onehot = (
        jax.lax.broadcasted_iota(jnp.int32, (TOK, BINS), 1) == idx[:, None]
    ).astype(jnp.bfloat16)
    def oh_dot(c_ref):
        return jax.lax.dot_general(
            onehot, c_ref[...], (((1,), (0,)), ((), ())),
            preferred_element_type=jnp.float32,
        )
    q = (oh_dot(c1_ref) + oh_dot(c2_ref)) + oh_dot(c3_ref)
    diff = q - r
    loss_ref[0:1, :] += jnp.sum(diff * diff)
    q_st = r + diff
    quant_out_ref[...] = quant_ref[...] + q_st
    r_out_ref[...] = r - q_st
    codes_ref[0, 0, :] = idx


def _stage(residual, rn, cb, c1, c2, c3, cbn, quant):
    return pl.pallas_call(
        _stage_kernel,
        grid=(NB,),
        in_specs=[
            pl.BlockSpec((TOK, DIM), lambda j: (j, 0)),
            pl.BlockSpec((1, 1, TOK), lambda j: (j, 0, 0)),
            pl.BlockSpec((BINS, DIM), lambda j: (0, 0)),
            pl.BlockSpec((BINS, DIM), lambda j: (0, 0)),
            pl.BlockSpec((BINS, DIM), lambda j: (0, 0)),
            pl.BlockSpec((BINS, DIM), lambda j: (0, 0)),
            pl.BlockSpec((1, BINS), lambda j: (0, 0)),
            pl.BlockSpec((TOK, DIM), lambda j: (j, 0)),
        ],
        out_specs=[
            pl.BlockSpec((TOK, DIM), lambda j: (j, 0)),
            pl.BlockSpec((TOK, DIM), lambda j: (j, 0)),
            pl.BlockSpec((1, 1, TOK), lambda j: (j, 0, 0)),
            pl.BlockSpec((8, 128), lambda j: (0, 0)),
        ],
        out_shape=[
            jax.ShapeDtypeStruct((NTOK, DIM), jnp.float32),
            jax.ShapeDtypeStruct((NTOK, DIM), jnp.float32),
            jax.ShapeDtypeStruct((NB, 1, TOK), jnp.int32),
            jax.ShapeDtypeStruct((8, 128), jnp.float32),
        ],
        input_output_aliases={0: 0, 7: 1},
    )(residual, rn, cb, c1, c2, c3, cbn, quant)


def _split_kernel(cb_ref, c1_ref, c2_ref, c3_ref):
    cb = cb_ref[...]
    c1 = cb.astype(jnp.bfloat16)
    rem1 = cb - c1.astype(jnp.float32)
    c2 = rem1.astype(jnp.bfloat16)
    c3 = (rem1 - c2.astype(jnp.float32)).astype(jnp.bfloat16)
    c1_ref[...] = c1
    c2_ref[...] = c2
    c3_ref[...] = c3


def kernel(x, codebooks):
    xt = jnp.transpose(x, (0, 2, 1))
    cbn = jnp.sum(codebooks * codebooks, axis=-1)
    c1, c2, c3 = pl.pallas_call(
        _split_kernel,
        out_shape=[
            jax.ShapeDtypeStruct((N_Q * BINS, DIM), jnp.bfloat16)
            for _ in range(3)
        ],
    )(codebooks.reshape(N_Q * BINS, DIM))
    c1 = c1.reshape(N_Q, BINS, DIM)
    c2 = c2.reshape(N_Q, BINS, DIM)
    c3 = c3.reshape(N_Q, BINS, DIM)

    residual = xt.reshape(NTOK, DIM)
    quant = jnp.zeros((NTOK, DIM), jnp.float32)
    codes_list = []
    loss_list = []
    for i in range(N_Q):
        rn = jnp.sum(
            residual.reshape(B, T, DIM) * residual.reshape(B, T, DIM),
            axis=-1, keepdims=True,
        ).reshape(NB, 1, TOK)
        residual, quant, codes_i, loss_i = _stage(
            residual, rn, codebooks[i], c1[i], c2[i], c3[i],
            cbn[i].reshape(1, BINS), quant
        )
        codes_list.append(codes_i.reshape(NTOK))
        loss_list.append(loss_i[0, 0])

    quantized = jnp.transpose(quant.reshape(B, T, DIM), (0, 2, 1))
    codes = jnp.stack(codes_list, axis=0).reshape(N_Q, B, T).astype(jnp.int64)
    commit_loss = jnp.mean(
        jnp.stack(loss_list) / jnp.float32(NTOK * DIM)
    )
    return quantized, codes, commit_loss


# speedup vs baseline: 1.6817x; 1.2500x over previous
import jax
import jax.numpy as jnp
from jax.experimental import pallas as pl
from jax.experimental.pallas import tpu as pltpu

N_Q = 8
BINS = 1024
DIM = 256
B = 8
T = 2048
NTOK = B * T
TOK = 2048
NB = NTOK // TOK
CHUNK = 1024
NCH = TOK // CHUNK


def _stage_kernel(r_ref, rn_ref, cb_ref, c1_ref, c2_ref, c3_ref, cbn_ref,
                  quant_ref, r_out_ref, quant_out_ref, codes_ref, loss_ref):
    j = pl.program_id(0)

    @pl.when(j == 0)
    def _init():
        loss_ref[...] = jnp.zeros_like(loss_ref)

    cb = cb_ref[...]
    cbn = cbn_ref[...]
    lsum = None
    for c in range(NCH):
        lo, hi = c * CHUNK, (c + 1) * CHUNK
        r = r_ref[lo:hi, :]
        rn = rn_ref[0][:, lo:hi]
        s = jax.lax.dot_general(
            r, cb, (((1,), (1,)), ((), ())),
            preferred_element_type=jnp.float32,
        )
        d = rn.reshape(CHUNK, 1) - 2.0 * s + cbn
        idx = jnp.argmin(d, axis=1)
        onehot = (
            jax.lax.broadcasted_iota(jnp.int32, (CHUNK, BINS), 1)
            == idx[:, None]
        ).astype(jnp.bfloat16)
        def oh_dot(c_ref):
            return jax.lax.dot_general(
                onehot, c_ref[...], (((1,), (0,)), ((), ())),
                preferred_element_type=jnp.float32,
            )
        q = (oh_dot(c1_ref) + oh_dot(c2_ref)) + oh_dot(c3_ref)
        diff = q - r
        part = jnp.sum(diff * diff)
        lsum = part if lsum is None else lsum + part
        q_st = r + diff
        quant_out_ref[lo:hi, :] = quant_ref[lo:hi, :] + q_st
        r_out_ref[lo:hi, :] = r - q_st
        codes_ref[0, 0, lo:hi] = idx
    loss_ref[0:1, :] += lsum


def _stage(residual, rn, cb, c1, c2, c3, cbn, quant):
    return pl.pallas_call(
        _stage_kernel,
        grid=(NB,),
        in_specs=[
            pl.BlockSpec((TOK, DIM), lambda j: (j, 0)),
            pl.BlockSpec((1, 1, TOK), lambda j: (j, 0, 0)),
            pl.BlockSpec((BINS, DIM), lambda j: (0, 0)),
            pl.BlockSpec((BINS, DIM), lambda j: (0, 0)),
            pl.BlockSpec((BINS, DIM), lambda j: (0, 0)),
            pl.BlockSpec((BINS, DIM), lambda j: (0, 0)),
            pl.BlockSpec((1, BINS), lambda j: (0, 0)),
            pl.BlockSpec((TOK, DIM), lambda j: (j, 0)),
        ],
        out_specs=[
            pl.BlockSpec((TOK, DIM), lambda j: (j, 0)),
            pl.BlockSpec((TOK, DIM), lambda j: (j, 0)),
            pl.BlockSpec((1, 1, TOK), lambda j: (j, 0, 0)),
            pl.BlockSpec((8, 128), lambda j: (0, 0)),
        ],
        out_shape=[
            jax.ShapeDtypeStruct((NTOK, DIM), jnp.float32),
            jax.ShapeDtypeStruct((NTOK, DIM), jnp.float32),
            jax.ShapeDtypeStruct((NB, 1, TOK), jnp.int32),
            jax.ShapeDtypeStruct((8, 128), jnp.float32),
        ],
        input_output_aliases={0: 0, 7: 1},
    )(residual, rn, cb, c1, c2, c3, cbn, quant)


def _split_kernel(cb_ref, c1_ref, c2_ref, c3_ref):
    cb = cb_ref[...]
    c1 = cb.astype(jnp.bfloat16)
    rem1 = cb - c1.astype(jnp.float32)
    c2 = rem1.astype(jnp.bfloat16)
    c3 = (rem1 - c2.astype(jnp.float32)).astype(jnp.bfloat16)
    c1_ref[...] = c1
    c2_ref[...] = c2
    c3_ref[...] = c3


def kernel(x, codebooks):
    xt = jnp.transpose(x, (0, 2, 1))
    cbn = jnp.sum(codebooks * codebooks, axis=-1)
    c1, c2, c3 = pl.pallas_call(
        _split_kernel,
        out_shape=[
            jax.ShapeDtypeStruct((N_Q * BINS, DIM), jnp.bfloat16)
            for _ in range(3)
        ],
    )(codebooks.reshape(N_Q * BINS, DIM))
    c1 = c1.reshape(N_Q, BINS, DIM)
    c2 = c2.reshape(N_Q, BINS, DIM)
    c3 = c3.reshape(N_Q, BINS, DIM)

    residual = xt.reshape(NTOK, DIM)
    quant = jnp.zeros((NTOK, DIM), jnp.float32)
    codes_list = []
    loss_list = []
    for i in range(N_Q):
        rn = jnp.sum(
            residual.reshape(B, T, DIM) * residual.reshape(B, T, DIM),
            axis=-1, keepdims=True,
        ).reshape(NB, 1, TOK)
        residual, quant, codes_i, loss_i = _stage(
            residual, rn, codebooks[i], c1[i], c2[i], c3[i],
            cbn[i].reshape(1, BINS), quant
        )
        codes_list.append(codes_i.reshape(NTOK))
        loss_list.append(loss_i[0, 0])

    quantized = jnp.transpose(quant.reshape(B, T, DIM), (0, 2, 1))
    codes = jnp.stack(codes_list, axis=0).reshape(N_Q, B, T).astype(jnp.int64)
    commit_loss = jnp.mean(
        jnp.stack(loss_list) / jnp.float32(NTOK * DIM)
    )
    return quantized, codes, commit_loss


# speedup vs baseline: 1.6849x; 1.0019x over previous
import jax
import jax.numpy as jnp
from jax.experimental import pallas as pl
from jax.experimental.pallas import tpu as pltpu

N_Q = 8
BINS = 1024
DIM = 256
B = 8
T = 2048
NTOK = B * T
TOK = 2048
NB = NTOK // TOK
CHUNK = 1024
NCH = TOK // CHUNK


def _stage_kernel(r_ref, rn_ref, cb_ref, c1_ref, c2_ref, c3_ref, cbn_ref,
                  quant_ref, r_out_ref, quant_out_ref, codes_ref, loss_ref):
    j = pl.program_id(0)

    @pl.when(j == 0)
    def _init():
        loss_ref[...] = jnp.zeros_like(loss_ref)

    cb = cb_ref[...]
    cbn = cbn_ref[...]
    lsum = None
    for c in range(NCH):
        lo, hi = c * CHUNK, (c + 1) * CHUNK
        r = r_ref[lo:hi, :]
        rn = rn_ref[0][:, lo:hi]
        s = jax.lax.dot_general(
            r, cb, (((1,), (1,)), ((), ())),
            preferred_element_type=jnp.float32,
        )
        d = rn.reshape(CHUNK, 1) - 2.0 * s + cbn
        iota = jax.lax.broadcasted_iota(jnp.int32, (CHUNK, BINS), 1)
        m = jnp.min(d, axis=1, keepdims=True)
        idx = jnp.min(
            jnp.where(d == m, iota, jnp.int32(BINS)), axis=1
        )
        onehot = (iota == idx[:, None]).astype(jnp.bfloat16)
        def oh_dot(c_ref):
            return jax.lax.dot_general(
                onehot, c_ref[...], (((1,), (0,)), ((), ())),
                preferred_element_type=jnp.float32,
            )
        q = (oh_dot(c1_ref) + oh_dot(c2_ref)) + oh_dot(c3_ref)
        diff = q - r
        part = jnp.sum(diff * diff)
        lsum = part if lsum is None else lsum + part
        q_st = r + diff
        quant_out_ref[lo:hi, :] = quant_ref[lo:hi, :] + q_st
        r_out_ref[lo:hi, :] = r - q_st
        codes_ref[0, 0, lo:hi] = idx
    loss_ref[0:1, :] += lsum


def _stage(residual, rn, cb, c1, c2, c3, cbn, quant):
    return pl.pallas_call(
        _stage_kernel,
        grid=(NB,),
        in_specs=[
            pl.BlockSpec((TOK, DIM), lambda j: (j, 0)),
            pl.BlockSpec((1, 1, TOK), lambda j: (j, 0, 0)),
            pl.BlockSpec((BINS, DIM), lambda j: (0, 0)),
            pl.BlockSpec((BINS, DIM), lambda j: (0, 0)),
            pl.BlockSpec((BINS, DIM), lambda j: (0, 0)),
            pl.BlockSpec((BINS, DIM), lambda j: (0, 0)),
            pl.BlockSpec((1, BINS), lambda j: (0, 0)),
            pl.BlockSpec((TOK, DIM), lambda j: (j, 0)),
        ],
        out_specs=[
            pl.BlockSpec((TOK, DIM), lambda j: (j, 0)),
            pl.BlockSpec((TOK, DIM), lambda j: (j, 0)),
            pl.BlockSpec((1, 1, TOK), lambda j: (j, 0, 0)),
            pl.BlockSpec((8, 128), lambda j: (0, 0)),
        ],
        out_shape=[
            jax.ShapeDtypeStruct((NTOK, DIM), jnp.float32),
            jax.ShapeDtypeStruct((NTOK, DIM), jnp.float32),
            jax.ShapeDtypeStruct((NB, 1, TOK), jnp.int32),
            jax.ShapeDtypeStruct((8, 128), jnp.float32),
        ],
        input_output_aliases={0: 0, 7: 1},
    )(residual, rn, cb, c1, c2, c3, cbn, quant)


def _split_kernel(cb_ref, c1_ref, c2_ref, c3_ref):
    cb = cb_ref[...]
    c1 = cb.astype(jnp.bfloat16)
    rem1 = cb - c1.astype(jnp.float32)
    c2 = rem1.astype(jnp.bfloat16)
    c3 = (rem1 - c2.astype(jnp.float32)).astype(jnp.bfloat16)
    c1_ref[...] = c1
    c2_ref[...] = c2
    c3_ref[...] = c3


def kernel(x, codebooks):
    xt = jnp.transpose(x, (0, 2, 1))
    cbn = jnp.sum(codebooks * codebooks, axis=-1)
    c1, c2, c3 = pl.pallas_call(
        _split_kernel,
        out_shape=[
            jax.ShapeDtypeStruct((N_Q * BINS, DIM), jnp.bfloat16)
            for _ in range(3)
        ],
    )(codebooks.reshape(N_Q * BINS, DIM))
    c1 = c1.reshape(N_Q, BINS, DIM)
    c2 = c2.reshape(N_Q, BINS, DIM)
    c3 = c3.reshape(N_Q, BINS, DIM)

    residual = xt.reshape(NTOK, DIM)
    quant = jnp.zeros((NTOK, DIM), jnp.float32)
    codes_list = []
    loss_list = []
    for i in range(N_Q):
        rn = jnp.sum(
            residual.reshape(B, T, DIM) * residual.reshape(B, T, DIM),
            axis=-1, keepdims=True,
        ).reshape(NB, 1, TOK)
        residual, quant, codes_i, loss_i = _stage(
            residual, rn, codebooks[i], c1[i], c2[i], c3[i],
            cbn[i].reshape(1, BINS), quant
        )
        codes_list.append(codes_i.reshape(NTOK))
        loss_list.append(loss_i[0, 0])

    quantized = jnp.transpose(quant.reshape(B, T, DIM), (0, 2, 1))
    codes = jnp.stack(codes_list, axis=0).reshape(N_Q, B, T).astype(jnp.int64)
    commit_loss = jnp.mean(
        jnp.stack(loss_list) / jnp.float32(NTOK * DIM)
    )
    return quantized, codes, commit_loss


# speedup vs baseline: 1.7642x; 1.0470x over previous
import jax
import jax.numpy as jnp
from jax.experimental import pallas as pl
from jax.experimental.pallas import tpu as pltpu

N_Q = 8
BINS = 1024
DIM = 256
B = 8
T = 2048
NTOK = B * T
TOK = 2048
NB = NTOK // TOK
CHUNK = 1024
NCH = TOK // CHUNK


def _stage_kernel(r_ref, rn_ref, cb_ref, c1_ref, c2_ref, c3_ref, cbn_ref,
                  quant_ref, r_out_ref, quant_out_ref, codes_ref, loss_ref):
    j = pl.program_id(0)

    @pl.when(j == 0)
    def _init():
        loss_ref[...] = jnp.zeros_like(loss_ref)

    cb = cb_ref[...]
    cbn = cbn_ref[...]
    lsum = None
    for c in range(NCH):
        lo, hi = c * CHUNK, (c + 1) * CHUNK
        r = r_ref[lo:hi, :]
        rn = rn_ref[0][:, lo:hi]
        s = jax.lax.dot_general(
            r, cb, (((1,), (1,)), ((), ())),
            preferred_element_type=jnp.float32,
        )
        d = rn.reshape(CHUNK, 1) - 2.0 * s + cbn
        iota = jax.lax.broadcasted_iota(
            jnp.int32, (CHUNK, BINS), 1
        ).astype(jnp.float32)
        m = jnp.min(d, axis=1, keepdims=True)
        idxf = jnp.min(
            jnp.where(d == m, iota, jnp.float32(BINS)), axis=1, keepdims=True
        )
        onehot = (iota == idxf).astype(jnp.bfloat16)
        idx = idxf[:, 0].astype(jnp.int32)
        def oh_dot(c_ref):
            return jax.lax.dot_general(
                onehot, c_ref[...], (((1,), (0,)), ((), ())),
                preferred_element_type=jnp.float32,
            )
        q = (oh_dot(c1_ref) + oh_dot(c2_ref)) + oh_dot(c3_ref)
        diff = q - r
        part = jnp.sum(diff * diff)
        lsum = part if lsum is None else lsum + part
        q_st = r + diff
        quant_out_ref[lo:hi, :] = quant_ref[lo:hi, :] + q_st
        r_out_ref[lo:hi, :] = r - q_st
        codes_ref[0, 0, lo:hi] = idx
    loss_ref[0:1, :] += lsum


def _stage(residual, rn, cb, c1, c2, c3, cbn, quant):
    return pl.pallas_call(
        _stage_kernel,
        grid=(NB,),
        in_specs=[
            pl.BlockSpec((TOK, DIM), lambda j: (j, 0)),
            pl.BlockSpec((1, 1, TOK), lambda j: (j, 0, 0)),
            pl.BlockSpec((BINS, DIM), lambda j: (0, 0)),
            pl.BlockSpec((BINS, DIM), lambda j: (0, 0)),
            pl.BlockSpec((BINS, DIM), lambda j: (0, 0)),
            pl.BlockSpec((BINS, DIM), lambda j: (0, 0)),
            pl.BlockSpec((1, BINS), lambda j: (0, 0)),
            pl.BlockSpec((TOK, DIM), lambda j: (j, 0)),
        ],
        out_specs=[
            pl.BlockSpec((TOK, DIM), lambda j: (j, 0)),
            pl.BlockSpec((TOK, DIM), lambda j: (j, 0)),
            pl.BlockSpec((1, 1, TOK), lambda j: (j, 0, 0)),
            pl.BlockSpec((8, 128), lambda j: (0, 0)),
        ],
        out_shape=[
            jax.ShapeDtypeStruct((NTOK, DIM), jnp.float32),
            jax.ShapeDtypeStruct((NTOK, DIM), jnp.float32),
            jax.ShapeDtypeStruct((NB, 1, TOK), jnp.int32),
            jax.ShapeDtypeStruct((8, 128), jnp.float32),
        ],
        input_output_aliases={0: 0, 7: 1},
    )(residual, rn, cb, c1, c2, c3, cbn, quant)


def _split_kernel(cb_ref, c1_ref, c2_ref, c3_ref):
    cb = cb_ref[...]
    c1 = cb.astype(jnp.bfloat16)
    rem1 = cb - c1.astype(jnp.float32)
    c2 = rem1.astype(jnp.bfloat16)
    c3 = (rem1 - c2.astype(jnp.float32)).astype(jnp.bfloat16)
    c1_ref[...] = c1
    c2_ref[...] = c2
    c3_ref[...] = c3


def kernel(x, codebooks):
    xt = jnp.transpose(x, (0, 2, 1))
    cbn = jnp.sum(codebooks * codebooks, axis=-1)
    c1, c2, c3 = pl.pallas_call(
        _split_kernel,
        out_shape=[
            jax.ShapeDtypeStruct((N_Q * BINS, DIM), jnp.bfloat16)
            for _ in range(3)
        ],
    )(codebooks.reshape(N_Q * BINS, DIM))
    c1 = c1.reshape(N_Q, BINS, DIM)
    c2 = c2.reshape(N_Q, BINS, DIM)
    c3 = c3.reshape(N_Q, BINS, DIM)

    residual = xt.reshape(NTOK, DIM)
    quant = jnp.zeros((NTOK, DIM), jnp.float32)
    codes_list = []
    loss_list = []
    for i in range(N_Q):
        rn = jnp.sum(
            residual.reshape(B, T, DIM) * residual.reshape(B, T, DIM),
            axis=-1, keepdims=True,
        ).reshape(NB, 1, TOK)
        residual, quant, codes_i, loss_i = _stage(
            residual, rn, codebooks[i], c1[i], c2[i], c3[i],
            cbn[i].reshape(1, BINS), quant
        )
        codes_list.append(codes_i.reshape(NTOK))
        loss_list.append(loss_i[0, 0])

    quantized = jnp.transpose(quant.reshape(B, T, DIM), (0, 2, 1))
    codes = jnp.stack(codes_list, axis=0).reshape(N_Q, B, T).astype(jnp.int64)
    commit_loss = jnp.mean(
        jnp.stack(loss_list) / jnp.float32(NTOK * DIM)
    )
    return quantized, codes, commit_loss


# speedup vs baseline: 1.7983x; 1.0193x over previous
import jax
import jax.numpy as jnp
from jax.experimental import pallas as pl
from jax.experimental.pallas import tpu as pltpu

N_Q = 8
BINS = 1024
DIM = 256
B = 8
T = 2048
NTOK = B * T
TOK = 2048
NB = NTOK // TOK
CHUNK = 512
NCH = TOK // CHUNK


def _stage_kernel(r_ref, rn_ref, cb_ref, c1_ref, c2_ref, c3_ref, cbn_ref,
                  quant_ref, r_out_ref, quant_out_ref, codes_ref, loss_ref):
    j = pl.program_id(0)

    @pl.when(j == 0)
    def _init():
        loss_ref[...] = jnp.zeros_like(loss_ref)

    cb = cb_ref[...]
    cbn = cbn_ref[...]
    lsum = None
    for c in range(NCH):
        lo, hi = c * CHUNK, (c + 1) * CHUNK
        r = r_ref[lo:hi, :]
        rn = rn_ref[0][:, lo:hi]
        s = jax.lax.dot_general(
            r, cb, (((1,), (1,)), ((), ())),
            preferred_element_type=jnp.float32,
        )
        d = rn.reshape(CHUNK, 1) - 2.0 * s + cbn
        iota = jax.lax.broadcasted_iota(
            jnp.int32, (CHUNK, BINS), 1
        ).astype(jnp.float32)
        m = jnp.min(d, axis=1, keepdims=True)
        idxf = jnp.min(
            jnp.where(d == m, iota, jnp.float32(BINS)), axis=1, keepdims=True
        )
        onehot = (iota == idxf).astype(jnp.bfloat16)
        idx = idxf[:, 0].astype(jnp.int32)
        def oh_dot(c_ref):
            return jax.lax.dot_general(
                onehot, c_ref[...], (((1,), (0,)), ((), ())),
                preferred_element_type=jnp.float32,
            )
        q = (oh_dot(c1_ref) + oh_dot(c2_ref)) + oh_dot(c3_ref)
        diff = q - r
        part = jnp.sum(diff * diff)
        lsum = part if lsum is None else lsum + part
        q_st = r + diff
        quant_out_ref[lo:hi, :] = quant_ref[lo:hi, :] + q_st
        r_out_ref[lo:hi, :] = r - q_st
        codes_ref[0, 0, lo:hi] = idx
    loss_ref[0:1, :] += lsum


def _stage(residual, rn, cb, c1, c2, c3, cbn, quant):
    return pl.pallas_call(
        _stage_kernel,
        grid=(NB,),
        in_specs=[
            pl.BlockSpec((TOK, DIM), lambda j: (j, 0)),
            pl.BlockSpec((1, 1, TOK), lambda j: (j, 0, 0)),
            pl.BlockSpec((BINS, DIM), lambda j: (0, 0)),
            pl.BlockSpec((BINS, DIM), lambda j: (0, 0)),
            pl.BlockSpec((BINS, DIM), lambda j: (0, 0)),
            pl.BlockSpec((BINS, DIM), lambda j: (0, 0)),
            pl.BlockSpec((1, BINS), lambda j: (0, 0)),
            pl.BlockSpec((TOK, DIM), lambda j: (j, 0)),
        ],
        out_specs=[
            pl.BlockSpec((TOK, DIM), lambda j: (j, 0)),
            pl.BlockSpec((TOK, DIM), lambda j: (j, 0)),
            pl.BlockSpec((1, 1, TOK), lambda j: (j, 0, 0)),
            pl.BlockSpec((8, 128), lambda j: (0, 0)),
        ],
        out_shape=[
            jax.ShapeDtypeStruct((NTOK, DIM), jnp.float32),
            jax.ShapeDtypeStruct((NTOK, DIM), jnp.float32),
            jax.ShapeDtypeStruct((NB, 1, TOK), jnp.int32),
            jax.ShapeDtypeStruct((8, 128), jnp.float32),
        ],
        input_output_aliases={0: 0, 7: 1},
    )(residual, rn, cb, c1, c2, c3, cbn, quant)


def _split_kernel(cb_ref, c1_ref, c2_ref, c3_ref):
    cb = cb_ref[...]
    c1 = cb.astype(jnp.bfloat16)
    rem1 = cb - c1.astype(jnp.float32)
    c2 = rem1.astype(jnp.bfloat16)
    c3 = (rem1 - c2.astype(jnp.float32)).astype(jnp.bfloat16)
    c1_ref[...] = c1
    c2_ref[...] = c2
    c3_ref[...] = c3


def kernel(x, codebooks):
    xt = jnp.transpose(x, (0, 2, 1))
    cbn = jnp.sum(codebooks * codebooks, axis=-1)
    c1, c2, c3 = pl.pallas_call(
        _split_kernel,
        out_shape=[
            jax.ShapeDtypeStruct((N_Q * BINS, DIM), jnp.bfloat16)
            for _ in range(3)
        ],
    )(codebooks.reshape(N_Q * BINS, DIM))
    c1 = c1.reshape(N_Q, BINS, DIM)
    c2 = c2.reshape(N_Q, BINS, DIM)
    c3 = c3.reshape(N_Q, BINS, DIM)

    residual = xt.reshape(NTOK, DIM)
    quant = jnp.zeros((NTOK, DIM), jnp.float32)
    codes_list = []
    loss_list = []
    for i in range(N_Q):
        rn = jnp.sum(
            residual.reshape(B, T, DIM) * residual.reshape(B, T, DIM),
            axis=-1, keepdims=True,
        ).reshape(NB, 1, TOK)
        residual, quant, codes_i, loss_i = _stage(
            residual, rn, codebooks[i], c1[i], c2[i], c3[i],
            cbn[i].reshape(1, BINS), quant
        )
        codes_list.append(codes_i.reshape(NTOK))
        loss_list.append(loss_i[0, 0])

    quantized = jnp.transpose(quant.reshape(B, T, DIM), (0, 2, 1))
    codes = jnp.stack(codes_list, axis=0).reshape(N_Q, B, T).astype(jnp.int64)
    commit_loss = jnp.mean(
        jnp.stack(loss_list) / jnp.float32(NTOK * DIM)
    )
    return quantized, codes, commit_loss
